# baseline (speedup 1.0000x reference)
import functools

import jax
import jax.numpy as jnp
from jax import lax
from jax.experimental import pallas as pl
from jax.experimental.pallas import tpu as pltpu
from jax.experimental.pallas import tpu_sc as plsc

EMBED_DIM = 16
BATCH = 16384
NTEAMS = 100000
NUM_CORES = 2
NUM_SUBCORES = 16
NUM_WORKERS = NUM_CORES * NUM_SUBCORES
BPW = BATCH // NUM_WORKERS
EPS = 1e-5


NQ = 8
NJ = EMBED_DIM // NQ


def _sc_gather_body(idx1_hbm, idx2_hbm, ttq_hbm, t1_hbm, t2_hbm,
                    idx1_v, idx2_v, idxs1_v, idxs2_v, q1_v, q2_v,
                    rows1_v, rows2_v, sem1, sem2):
    wid = lax.axis_index("s") * NUM_CORES + lax.axis_index("c")
    base = wid * BPW
    pltpu.sync_copy(idx1_hbm.at[pl.ds(base, BPW)], idx1_v)
    pltpu.sync_copy(idx2_hbm.at[pl.ds(base, BPW)], idx2_v)

    def build(k, _):
        v1 = idx1_v[pl.ds(k * 16, 16)]
        v2 = idx2_v[pl.ds(k * 16, 16)]
        for j in range(NJ):
            idxs1_v[j, pl.ds(k * 16, 16)] = v1 + (j * NTEAMS)
            idxs2_v[j, pl.ds(k * 16, 16)] = v2 + (j * NTEAMS)
        return 0

    lax.fori_loop(0, BPW // 16, build, 0, unroll=2)

    copies = []
    for j in range(NJ):
        copies.append(
            pltpu.async_copy(ttq_hbm.at[idxs1_v.at[j]], q1_v.at[j], sem1))
        copies.append(
            pltpu.async_copy(ttq_hbm.at[idxs2_v.at[j]], q2_v.at[j], sem2))
    for cp in copies:
        cp.wait()

    lanes = lax.iota(jnp.int32, 16)

    def rearrange(blk, _):
        p = blk * 16 + lanes
        for j in range(NJ):
            jj = jnp.full((16,), j, jnp.int32)
            for k in range(NQ):
                kk = jnp.full((16,), k, jnp.int32)
                rows1_v[NQ * j + k, pl.ds(blk * 16, 16)] = (
                    plsc.load_gather(q1_v, [jj, p, kk]))
                rows2_v[NQ * j + k, pl.ds(blk * 16, 16)] = (
                    plsc.load_gather(q2_v, [jj, p, kk]))
        return 0

    lax.fori_loop(0, BPW // 16, rearrange, 0, unroll=2)
    pltpu.sync_copy(rows1_v, t1_hbm.at[:, pl.ds(base, BPW)])
    pltpu.sync_copy(rows2_v, t2_hbm.at[:, pl.ds(base, BPW)])


@jax.jit
def _sc_gather(idx1, idx2, ttq):
    mesh = plsc.VectorSubcoreMesh(core_axis_name="c", subcore_axis_name="s")
    fn = functools.partial(
        pl.kernel,
        mesh=mesh,
        out_type=[
            jax.ShapeDtypeStruct((EMBED_DIM, BATCH), jnp.float32),
            jax.ShapeDtypeStruct((EMBED_DIM, BATCH), jnp.float32),
        ],
        scratch_types=[
            pltpu.VMEM((BPW,), jnp.int32),
            pltpu.VMEM((BPW,), jnp.int32),
            pltpu.VMEM((NJ, BPW), jnp.int32),
            pltpu.VMEM((NJ, BPW), jnp.int32),
            pltpu.VMEM((NJ, BPW, NQ), jnp.float32),
            pltpu.VMEM((NJ, BPW, NQ), jnp.float32),
            pltpu.VMEM((EMBED_DIM, BPW), jnp.float32),
            pltpu.VMEM((EMBED_DIM, BPW), jnp.float32),
            pltpu.SemaphoreType.DMA,
            pltpu.SemaphoreType.DMA,
        ],
        compiler_params=pltpu.CompilerParams(use_tc_tiling_on_sc=False,
                                             needs_layout_passes=False),
    )(_sc_gather_body)
    return fn(idx1, idx2, ttq)


def _tc_classifier_body(t1_ref, t2_ref, sd_ref, par_ref, scal_ref, out_ref):
    inv_b = 1.0 / BATCH
    t1 = t1_ref[...].reshape(EMBED_DIM, 128, 128)
    t2 = t2_ref[...].reshape(EMBED_DIM, 128, 128)
    sd = sd_ref[...]

    m1 = jnp.sum(t1, axis=(1, 2), keepdims=True) * inv_b
    m2 = jnp.sum(t2, axis=(1, 2), keepdims=True) * inv_b
    c1 = t1 - m1
    c2 = t2 - m2
    v1 = jnp.sum(c1 * c1, axis=(1, 2), keepdims=True) * inv_b
    v2 = jnp.sum(c2 * c2, axis=(1, 2), keepdims=True) * inv_b

    par = par_ref[...]
    g1 = par[:, 0:1].reshape(EMBED_DIM, 1, 1)
    g2 = par[:, 1:2].reshape(EMBED_DIM, 1, 1)
    b1 = par[:, 2:3]
    b2 = par[:, 3:4]
    w1 = par[:, 4:5].reshape(EMBED_DIM, 1, 1)
    w2 = par[:, 5:6].reshape(EMBED_DIM, 1, 1)
    gsd = scal_ref[0, 0]
    bsd = scal_ref[0, 1]
    wsd = scal_ref[0, 2]
    bias = scal_ref[0, 3]

    sw1 = g1 * jax.lax.rsqrt(v1 + EPS) * w1
    sw2 = g2 * jax.lax.rsqrt(v2 + EPS) * w2

    msd = jnp.sum(sd) * inv_b
    csd = sd - msd
    vsd = jnp.sum(csd * csd) * inv_b
    swsd = gsd * jax.lax.rsqrt(vsd + EPS) * wsd

    const = (jnp.sum(b1 * par[:, 4:5]) + jnp.sum(b2 * par[:, 5:6])
             + bsd * wsd + bias)
    logits = (jnp.sum(c1 * sw1, axis=0) + jnp.sum(c2 * sw2, axis=0)
              + csd * swsd + const)
    out_ref[...] = 1.0 / (1.0 + jnp.exp(-logits))


@jax.jit
def _tc_classifier(t1p, t2p, sd, par, scal):
    return pl.pallas_call(
        _tc_classifier_body,
        out_shape=jax.ShapeDtypeStruct((128, 128), jnp.float32),
    )(t1p, t2p, sd, par, scal)


def kernel(idsTensor, table, gamma, beta, W, b):
    idx1 = idsTensor[:, 0].astype(jnp.int32)
    idx2 = idsTensor[:, 1].astype(jnp.int32)
    sd = idsTensor[:, 2].reshape(128, 128)
    ttq = jnp.transpose(table.reshape(NTEAMS, NJ, NQ),
                        (1, 0, 2)).reshape(NJ * NTEAMS, NQ)
    t1, t2 = _sc_gather(idx1, idx2, ttq)
    t1p = t1.reshape(2048, 128)
    t2p = t2.reshape(2048, 128)
    par = jnp.stack(
        [gamma[:EMBED_DIM], gamma[EMBED_DIM:2 * EMBED_DIM],
         beta[:EMBED_DIM], beta[EMBED_DIM:2 * EMBED_DIM],
         W[0, :EMBED_DIM], W[0, EMBED_DIM:2 * EMBED_DIM]], axis=1)
    scal = jnp.stack(
        [gamma[2 * EMBED_DIM], beta[2 * EMBED_DIM], W[0, 2 * EMBED_DIM],
         b[0]]).reshape(1, 4)
    out = _tc_classifier(t1p, t2p, sd, par, scal)
    return out.reshape(BATCH, 1)


# speedup vs baseline: 2.1632x; 2.1632x over previous
import functools

import jax
import jax.numpy as jnp
from jax import lax
from jax.experimental import pallas as pl
from jax.experimental.pallas import tpu as pltpu
from jax.experimental.pallas import tpu_sc as plsc

EMBED_DIM = 16
BATCH = 16384
NTEAMS = 100000
NUM_CORES = 2
NUM_SUBCORES = 16
NUM_WORKERS = NUM_CORES * NUM_SUBCORES
BPW = BATCH // NUM_WORKERS
EPS = 1e-5


def _sc_gather_body(idx1_hbm, idx2_hbm, ttf_hbm, t1_hbm, t2_hbm,
                    idx1_v, idx2_v, idxs1_v, idxs2_v, rows1_v, rows2_v,
                    sem1, sem2):
    wid = lax.axis_index("s") * NUM_CORES + lax.axis_index("c")
    base = wid * BPW
    pltpu.sync_copy(idx1_hbm.at[pl.ds(base, BPW)], idx1_v)
    pltpu.sync_copy(idx2_hbm.at[pl.ds(base, BPW)], idx2_v)

    def build(k, _):
        v1 = idx1_v[pl.ds(k * 16, 16)]
        v2 = idx2_v[pl.ds(k * 16, 16)]
        for j in range(EMBED_DIM):
            idxs1_v[pl.ds(j * BPW + k * 16, 16)] = v1 + (j * NTEAMS)
            idxs2_v[pl.ds(j * BPW + k * 16, 16)] = v2 + (j * NTEAMS)
        return 0

    lax.fori_loop(0, BPW // 16, build, 0, unroll=2)

    cp1 = pltpu.async_copy(ttf_hbm.at[idxs1_v], rows1_v, sem1)
    cp2 = pltpu.async_copy(ttf_hbm.at[idxs2_v], rows2_v, sem2)
    cp1.wait()
    for j in range(EMBED_DIM):
        pltpu.sync_copy(rows1_v.at[pl.ds(j * BPW, BPW)],
                        t1_hbm.at[pl.ds(j * BATCH + base, BPW)])
    cp2.wait()
    for j in range(EMBED_DIM):
        pltpu.sync_copy(rows2_v.at[pl.ds(j * BPW, BPW)],
                        t2_hbm.at[pl.ds(j * BATCH + base, BPW)])


@jax.jit
def _sc_gather(idx1, idx2, ttf):
    mesh = plsc.VectorSubcoreMesh(core_axis_name="c", subcore_axis_name="s")
    fn = functools.partial(
        pl.kernel,
        mesh=mesh,
        out_type=[
            jax.ShapeDtypeStruct((EMBED_DIM * BATCH,), jnp.float32),
            jax.ShapeDtypeStruct((EMBED_DIM * BATCH,), jnp.float32),
        ],
        scratch_types=[
            pltpu.VMEM((BPW,), jnp.int32),
            pltpu.VMEM((BPW,), jnp.int32),
            pltpu.VMEM((EMBED_DIM * BPW,), jnp.int32),
            pltpu.VMEM((EMBED_DIM * BPW,), jnp.int32),
            pltpu.VMEM((EMBED_DIM * BPW,), jnp.float32),
            pltpu.VMEM((EMBED_DIM * BPW,), jnp.float32),
            pltpu.SemaphoreType.DMA,
            pltpu.SemaphoreType.DMA,
        ],
        compiler_params=pltpu.CompilerParams(use_tc_tiling_on_sc=False,
                                             needs_layout_passes=False),
    )(_sc_gather_body)
    return fn(idx1, idx2, ttf)


def _tc_classifier_body(t1_ref, t2_ref, sd_ref, par_ref, scal_ref, out_ref):
    inv_b = 1.0 / BATCH
    t1 = t1_ref[...].reshape(EMBED_DIM, 128, 128)
    t2 = t2_ref[...].reshape(EMBED_DIM, 128, 128)
    sd = sd_ref[...]

    m1 = jnp.sum(t1, axis=(1, 2), keepdims=True) * inv_b
    m2 = jnp.sum(t2, axis=(1, 2), keepdims=True) * inv_b
    c1 = t1 - m1
    c2 = t2 - m2
    v1 = jnp.sum(c1 * c1, axis=(1, 2), keepdims=True) * inv_b
    v2 = jnp.sum(c2 * c2, axis=(1, 2), keepdims=True) * inv_b

    par = par_ref[...]
    g1 = par[:, 0:1].reshape(EMBED_DIM, 1, 1)
    g2 = par[:, 1:2].reshape(EMBED_DIM, 1, 1)
    b1 = par[:, 2:3]
    b2 = par[:, 3:4]
    w1 = par[:, 4:5].reshape(EMBED_DIM, 1, 1)
    w2 = par[:, 5:6].reshape(EMBED_DIM, 1, 1)
    gsd = scal_ref[0, 0]
    bsd = scal_ref[0, 1]
    wsd = scal_ref[0, 2]
    bias = scal_ref[0, 3]

    sw1 = g1 * jax.lax.rsqrt(v1 + EPS) * w1
    sw2 = g2 * jax.lax.rsqrt(v2 + EPS) * w2

    msd = jnp.sum(sd) * inv_b
    csd = sd - msd
    vsd = jnp.sum(csd * csd) * inv_b
    swsd = gsd * jax.lax.rsqrt(vsd + EPS) * wsd

    const = (jnp.sum(b1 * par[:, 4:5]) + jnp.sum(b2 * par[:, 5:6])
             + bsd * wsd + bias)
    logits = (jnp.sum(c1 * sw1, axis=0) + jnp.sum(c2 * sw2, axis=0)
              + csd * swsd + const)
    out_ref[...] = 1.0 / (1.0 + jnp.exp(-logits))


@jax.jit
def _tc_classifier(t1p, t2p, sd, par, scal):
    return pl.pallas_call(
        _tc_classifier_body,
        out_shape=jax.ShapeDtypeStruct((128, 128), jnp.float32),
    )(t1p, t2p, sd, par, scal)


def kernel(idsTensor, table, gamma, beta, W, b):
    idx1 = idsTensor[:, 0].astype(jnp.int32)
    idx2 = idsTensor[:, 1].astype(jnp.int32)
    sd = idsTensor[:, 2].reshape(128, 128)
    ttf = table.T.reshape(EMBED_DIM * NTEAMS)
    t1, t2 = _sc_gather(idx1, idx2, ttf)
    t1p = t1.reshape(2048, 128)
    t2p = t2.reshape(2048, 128)
    par = jnp.stack(
        [gamma[:EMBED_DIM], gamma[EMBED_DIM:2 * EMBED_DIM],
         beta[:EMBED_DIM], beta[EMBED_DIM:2 * EMBED_DIM],
         W[0, :EMBED_DIM], W[0, EMBED_DIM:2 * EMBED_DIM]], axis=1)
    scal = jnp.stack(
        [gamma[2 * EMBED_DIM], beta[2 * EMBED_DIM], W[0, 2 * EMBED_DIM],
         b[0]]).reshape(1, 4)
    out = _tc_classifier(t1p, t2p, sd, par, scal)
    return out.reshape(BATCH, 1)


# speedup vs baseline: 2.2284x; 1.0302x over previous
import functools

import jax
import jax.numpy as jnp
from jax import lax
from jax.experimental import pallas as pl
from jax.experimental.pallas import tpu as pltpu
from jax.experimental.pallas import tpu_sc as plsc

EMBED_DIM = 16
BATCH = 16384
NTEAMS = 100000
NUM_CORES = 2
NUM_SUBCORES = 16
NUM_WORKERS = NUM_CORES * NUM_SUBCORES
BPW = BATCH // NUM_WORKERS
EPS = 1e-5


def _sc_gather_body(idx1_hbm, idx2_hbm, ttf_hbm, t1_hbm, t2_hbm,
                    idx1_v, idx2_v, idxs1_v, idxs2_v, rows1_v, rows2_v,
                    sem1, sem2):
    wid = lax.axis_index("s") * NUM_CORES + lax.axis_index("c")
    base = wid * BPW
    pltpu.sync_copy(idx1_hbm.at[pl.ds(base, BPW)], idx1_v)
    pltpu.sync_copy(idx2_hbm.at[pl.ds(base, BPW)], idx2_v)

    def build(k, _):
        v1 = idx1_v[pl.ds(k * 16, 16)]
        v2 = idx2_v[pl.ds(k * 16, 16)]
        for j in range(EMBED_DIM):
            idxs1_v[j, pl.ds(k * 16, 16)] = v1 + (j * NTEAMS)
            idxs2_v[j, pl.ds(k * 16, 16)] = v2 + (j * NTEAMS)
        return 0

    lax.fori_loop(0, BPW // 16, build, 0, unroll=2)

    copies = []
    for j in range(EMBED_DIM):
        copies.append(
            pltpu.async_copy(ttf_hbm.at[idxs1_v.at[j]], rows1_v.at[j], sem1))
        copies.append(
            pltpu.async_copy(ttf_hbm.at[idxs2_v.at[j]], rows2_v.at[j], sem2))
    for cp in copies:
        cp.wait()
    pltpu.sync_copy(rows1_v, t1_hbm.at[:, pl.ds(base, BPW)])
    pltpu.sync_copy(rows2_v, t2_hbm.at[:, pl.ds(base, BPW)])


@jax.jit
def _sc_gather(idx1, idx2, ttf):
    mesh = plsc.VectorSubcoreMesh(core_axis_name="c", subcore_axis_name="s")
    fn = functools.partial(
        pl.kernel,
        mesh=mesh,
        out_type=[
            jax.ShapeDtypeStruct((EMBED_DIM, BATCH), jnp.float32),
            jax.ShapeDtypeStruct((EMBED_DIM, BATCH), jnp.float32),
        ],
        scratch_types=[
            pltpu.VMEM((BPW,), jnp.int32),
            pltpu.VMEM((BPW,), jnp.int32),
            pltpu.VMEM((EMBED_DIM, BPW), jnp.int32),
            pltpu.VMEM((EMBED_DIM, BPW), jnp.int32),
            pltpu.VMEM((EMBED_DIM, BPW), jnp.float32),
            pltpu.VMEM((EMBED_DIM, BPW), jnp.float32),
            pltpu.SemaphoreType.DMA,
            pltpu.SemaphoreType.DMA,
        ],
        compiler_params=pltpu.CompilerParams(use_tc_tiling_on_sc=False,
                                             needs_layout_passes=False),
    )(_sc_gather_body)
    return fn(idx1, idx2, ttf)


def _tc_classifier_body(t1_ref, t2_ref, sd_ref, par_ref, scal_ref, out_ref):
    inv_b = 1.0 / BATCH
    t1 = t1_ref[...].reshape(EMBED_DIM, 128, 128)
    t2 = t2_ref[...].reshape(EMBED_DIM, 128, 128)
    sd = sd_ref[...]

    m1 = jnp.sum(t1, axis=(1, 2), keepdims=True) * inv_b
    m2 = jnp.sum(t2, axis=(1, 2), keepdims=True) * inv_b
    c1 = t1 - m1
    c2 = t2 - m2
    v1 = jnp.sum(c1 * c1, axis=(1, 2), keepdims=True) * inv_b
    v2 = jnp.sum(c2 * c2, axis=(1, 2), keepdims=True) * inv_b

    par = par_ref[...]
    g1 = par[:, 0:1].reshape(EMBED_DIM, 1, 1)
    g2 = par[:, 1:2].reshape(EMBED_DIM, 1, 1)
    b1 = par[:, 2:3]
    b2 = par[:, 3:4]
    w1 = par[:, 4:5].reshape(EMBED_DIM, 1, 1)
    w2 = par[:, 5:6].reshape(EMBED_DIM, 1, 1)
    gsd = scal_ref[0, 0]
    bsd = scal_ref[0, 1]
    wsd = scal_ref[0, 2]
    bias = scal_ref[0, 3]

    sw1 = g1 * jax.lax.rsqrt(v1 + EPS) * w1
    sw2 = g2 * jax.lax.rsqrt(v2 + EPS) * w2

    msd = jnp.sum(sd) * inv_b
    csd = sd - msd
    vsd = jnp.sum(csd * csd) * inv_b
    swsd = gsd * jax.lax.rsqrt(vsd + EPS) * wsd

    const = (jnp.sum(b1 * par[:, 4:5]) + jnp.sum(b2 * par[:, 5:6])
             + bsd * wsd + bias)
    logits = (jnp.sum(c1 * sw1, axis=0) + jnp.sum(c2 * sw2, axis=0)
              + csd * swsd + const)
    out_ref[...] = 1.0 / (1.0 + jnp.exp(-logits))


@jax.jit
def _tc_classifier(t1p, t2p, sd, par, scal):
    return pl.pallas_call(
        _tc_classifier_body,
        out_shape=jax.ShapeDtypeStruct((128, 128), jnp.float32),
    )(t1p, t2p, sd, par, scal)


def kernel(idsTensor, table, gamma, beta, W, b):
    idx1 = idsTensor[:, 0].astype(jnp.int32)
    idx2 = idsTensor[:, 1].astype(jnp.int32)
    sd = idsTensor[:, 2].reshape(128, 128)
    ttf = table.T.reshape(EMBED_DIM * NTEAMS)
    t1, t2 = _sc_gather(idx1, idx2, ttf)
    t1p = t1.reshape(2048, 128)
    t2p = t2.reshape(2048, 128)
    par = jnp.stack(
        [gamma[:EMBED_DIM], gamma[EMBED_DIM:2 * EMBED_DIM],
         beta[:EMBED_DIM], beta[EMBED_DIM:2 * EMBED_DIM],
         W[0, :EMBED_DIM], W[0, EMBED_DIM:2 * EMBED_DIM]], axis=1)
    scal = jnp.stack(
        [gamma[2 * EMBED_DIM], beta[2 * EMBED_DIM], W[0, 2 * EMBED_DIM],
         b[0]]).reshape(1, 4)
    out = _tc_classifier(t1p, t2p, sd, par, scal)
    return out.reshape(BATCH, 1)
